# Initial kernel scaffold; baseline (speedup 1.0000x reference)
import jax
import jax.numpy as jnp
from jax.experimental import pallas as pl


def kernel(user_embeddings, item_embeddings, adj_src, adj_tgt, tpadj_src, tpadj_tgt):
    raise NotImplementedError("write your pallas kernel here")


# speedup vs baseline: 5.1879x; 5.1879x over previous
import functools

import jax
import jax.numpy as jnp
from jax import lax
from jax.experimental import pallas as pl
from jax.experimental.pallas import tpu as pltpu
from jax.experimental.pallas import tpu_sc as plsc

N = 50000
D = 128
E = 600000
GRAPHNUM = 2
NCHUNK = 10
R = N // NCHUNK
BR = 40
NBLK = R // BR
EB = 128
NSUB = 16
NLANE = 16
LEAK = 0.01


def _layer_body(i_tab, u_tab, a_src, a_tgt, t_src, t_tgt, a_off, t_off,
                vec_u_in, vec_i_in,
                u_new, i_new, vu_out, vi_out,
                acc, offs, sidx, tidx, rows, zb, ob_acc, ob_prev, ob_vec, sem):
    c = lax.axis_index("c")
    s = lax.axis_index("s")

    @pl.loop(0, BR)
    def _(r):
        for j in range(D // NLANE):
            zb[r, pl.ds(NLANE * j, NLANE)] = jnp.zeros((NLANE,), jnp.float32)

    def do_direction(src_tab, e_src, e_tgt, off_hbm, prev_tab, vec_in,
                     tab_out, vec_out):
        pltpu.sync_copy(off_hbm, offs)
        ovec = offs[pl.ds(0, 16)]

        for slot in range(NCHUNK // 2):
            chunk = 2 * slot + c
            base = chunk * R
            lo = jnp.where(c == 0, ovec[2 * slot], ovec[2 * slot + 1])
            hi = jnp.where(c == 0, ovec[2 * slot + 1], ovec[2 * slot + 2])

            @pl.loop(s, NBLK, step=NSUB)
            def _(b):
                pltpu.sync_copy(zb, acc.at[pl.ds(b * BR, BR)])

            plsc.subcore_barrier()

            n = hi - lo
            t_lo = lo + (n * s) // NSUB
            t_hi = lo + (n * (s + 1)) // NSUB
            a_lo = (t_lo // 16) * 16

            @pl.loop(a_lo, t_hi, step=EB)
            def _(e):
                pltpu.sync_copy(e_src.at[pl.ds(e, EB)], sidx)
                pltpu.sync_copy(e_tgt.at[pl.ds(e, EB)], tidx)
                for j in range(EB // NLANE):
                    sl = pl.ds(NLANE * j, NLANE)
                    pos = e + NLANE * j + lax.iota(jnp.int32, 16)
                    valid = (pos >= t_lo) & (pos < t_hi)
                    tidx[sl] = jnp.where(valid, tidx[sl] - base, R)
                pltpu.async_copy(src_tab.at[sidx], rows, sem).wait()
                pltpu.sync_copy(rows, acc.at[tidx], add=True)

            plsc.subcore_barrier()

            @pl.loop(s, NBLK, step=NSUB)
            def _(b):
                r0 = b * BR
                g0 = base + r0
                pltpu.sync_copy(acc.at[pl.ds(r0, BR)], ob_acc)
                pltpu.sync_copy(prev_tab.at[pl.ds(g0, BR)], ob_prev)
                pltpu.sync_copy(vec_in.at[pl.ds(g0, BR)], ob_vec)

                @pl.loop(0, BR)
                def _(r):
                    for j in range(D // NLANE):
                        sl = pl.ds(NLANE * j, NLANE)
                        x = ob_acc[r, sl]
                        y = jnp.maximum(x, LEAK * x) + ob_prev[r, sl]
                        ob_prev[r, sl] = y
                        ob_vec[r, sl] = ob_vec[r, sl] + y

                pltpu.sync_copy(ob_prev, tab_out.at[pl.ds(g0, BR)])
                pltpu.sync_copy(ob_vec, vec_out.at[pl.ds(g0, BR)])

            plsc.subcore_barrier()

    do_direction(i_tab, a_src, a_tgt, a_off, u_tab, vec_u_in, u_new, vu_out)
    do_direction(u_tab, t_src, t_tgt, t_off, i_tab, vec_i_in, i_new, vi_out)


_tab = jax.ShapeDtypeStruct((N, D), jnp.float32)

_layer = pl.kernel(
    _layer_body,
    out_type=(_tab, _tab, _tab, _tab),
    mesh=plsc.VectorSubcoreMesh(core_axis_name="c", subcore_axis_name="s"),
    scratch_types=[
        pltpu.VMEM_SHARED((R + 8, D), jnp.float32),
        pltpu.VMEM((16,), jnp.int32),
        pltpu.VMEM((EB,), jnp.int32),
        pltpu.VMEM((EB,), jnp.int32),
        pltpu.VMEM((EB, D), jnp.float32),
        pltpu.VMEM((BR, D), jnp.float32),
        pltpu.VMEM((BR, D), jnp.float32),
        pltpu.VMEM((BR, D), jnp.float32),
        pltpu.VMEM((BR, D), jnp.float32),
        pltpu.SemaphoreType.DMA,
    ],
)


def _mk_off(tgt):
    bounds = (jnp.arange(1, NCHUNK, dtype=jnp.int32) * R)
    b = jnp.searchsorted(tgt, bounds, side="left").astype(jnp.int32)
    return jnp.concatenate([
        jnp.zeros((1,), jnp.int32), b, jnp.full((1,), E, jnp.int32),
        jnp.zeros((16 - NCHUNK - 1,), jnp.int32)])


def kernel(user_embeddings, item_embeddings, adj_src, adj_tgt,
           tpadj_src, tpadj_tgt):
    user_vecs = []
    item_vecs = []
    for k in range(GRAPHNUM):
        u0 = user_embeddings[k]
        i0 = item_embeddings[k]
        asrc = jnp.pad(adj_src[k], (0, EB))
        atgt = jnp.pad(adj_tgt[k], (0, EB))
        tsrc = jnp.pad(tpadj_src[k], (0, EB))
        ttgt = jnp.pad(tpadj_tgt[k], (0, EB))
        a_off = _mk_off(adj_tgt[k])
        t_off = _mk_off(tpadj_tgt[k])
        u1, i1, vu1, vi1 = _layer(i0, u0, asrc, atgt, tsrc, ttgt,
                                  a_off, t_off, u0, i0)
        u2, i2, vu2, vi2 = _layer(i1, u1, asrc, atgt, tsrc, ttgt,
                                  a_off, t_off, vu1, vi1)
        user_vecs.append(vu2)
        item_vecs.append(vi2)
    return (jnp.stack(user_vecs, axis=0), jnp.stack(item_vecs, axis=0))


# speedup vs baseline: 9.9387x; 1.9157x over previous
import functools

import jax
import jax.numpy as jnp
from jax import lax
from jax.experimental import pallas as pl
from jax.experimental.pallas import tpu as pltpu
from jax.experimental.pallas import tpu_sc as plsc

N = 50000
D = 128
E = 600000
GRAPHNUM = 2
NCHUNK = 10
R = N // NCHUNK
BR = 40
NBLK = R // BR
EB = 128
SUP = 16
EPAD = SUP * EB
NSUB = 16
NLANE = 16
LEAK = 0.01


def _layer_body(first_layer, last_layer,
                i_tab, u_tab, a_src, a_tgt, t_src, t_tgt, a_off, t_off,
                vec_u_in, vec_i_in,
                u_new, i_new, vu_out, vi_out,
                acc, offs, sbulk, tbulk, ssm0, ssm1, tsm0, tsm1,
                rows0, rows1, zb, ob_acc, ob_prev, ob_vec,
                semb, sem0, sem1):
    c = lax.axis_index("c")
    s = lax.axis_index("s")
    ssm = (ssm0, ssm1)
    tsm = (tsm0, tsm1)
    rows = (rows0, rows1)
    sems = (sem0, sem1)

    @pl.loop(0, BR)
    def _(r):
        for j in range(D // NLANE):
            zb[r, pl.ds(NLANE * j, NLANE)] = jnp.zeros((NLANE,), jnp.float32)

    def do_direction(src_tab, e_src, e_tgt, off_hbm, prev_tab, vec_in,
                     tab_out, vec_out):
        pltpu.sync_copy(off_hbm, offs)
        ovec = offs[pl.ds(0, 16)]

        for slot in range(NCHUNK // 2):
            chunk = 2 * slot + c
            base = chunk * R
            lo = jnp.where(c == 0, ovec[2 * slot], ovec[2 * slot + 1])
            hi = jnp.where(c == 0, ovec[2 * slot + 1], ovec[2 * slot + 2])

            @pl.loop(s, NBLK, step=NSUB)
            def _(b):
                pltpu.sync_copy(zb, acc.at[pl.ds(b * BR, BR)])

            plsc.subcore_barrier()

            n = hi - lo
            t_lo = lo + (n * s) // NSUB
            t_hi = lo + (n * (s + 1)) // NSUB
            a_lo = (t_lo // 16) * 16
            nb = (t_hi - a_lo + EB - 1) // EB
            nb = jnp.maximum(nb, 0)

            def bulk_load(bi):
                e = a_lo + bi * EB
                d1 = pltpu.async_copy(e_src.at[pl.ds(e, SUP * EB)], sbulk,
                                      semb)
                d2 = pltpu.async_copy(e_tgt.at[pl.ds(e, SUP * EB)], tbulk,
                                      semb)
                d1.wait()
                d2.wait()

            def stage(bi, par):
                q = lax.rem(bi, SUP)
                e = a_lo + bi * EB
                for j in range(EB // NLANE):
                    sl = pl.ds(NLANE * j, NLANE)
                    bsl = pl.ds(q * EB + NLANE * j, NLANE)
                    pos = e + NLANE * j + lax.iota(jnp.int32, 16)
                    valid = (pos >= t_lo) & (pos < t_hi)
                    ssm[par][sl] = sbulk[bsl]
                    tsm[par][sl] = jnp.where(valid, tbulk[bsl] - base, R)
                pltpu.async_copy(src_tab.at[ssm[par]], rows[par], sems[par])

            @pl.when(nb > 0)
            def _():
                bulk_load(0)
                stage(0, 0)

            @pl.loop(0, (nb + 1) // 2)
            def _(h):
                for par in range(2):
                    bi = 2 * h + par

                    @pl.when(bi < nb)
                    def _():
                        nxt = bi + 1

                        @pl.when(nxt < nb)
                        def _():
                            @pl.when(lax.rem(nxt, SUP) == 0)
                            def _():
                                bulk_load(nxt)

                            stage(nxt, 1 - par)

                        pltpu.make_async_copy(src_tab.at[ssm[par]],
                                              rows[par], sems[par]).wait()
                        pltpu.sync_copy(rows[par], acc.at[tsm[par]], add=True)

            plsc.subcore_barrier()

            @pl.loop(s, NBLK, step=NSUB)
            def _(b):
                r0 = b * BR
                g0 = base + r0
                d1 = pltpu.async_copy(acc.at[pl.ds(r0, BR)], ob_acc, sem0)
                d2 = pltpu.async_copy(prev_tab.at[pl.ds(g0, BR)], ob_prev,
                                      sem1)
                if not first_layer:
                    d3 = pltpu.async_copy(vec_in.at[pl.ds(g0, BR)], ob_vec,
                                          semb)
                    d3.wait()
                d1.wait()
                d2.wait()

                @pl.loop(0, BR)
                def _(r):
                    for j in range(D // NLANE):
                        sl = pl.ds(NLANE * j, NLANE)
                        x = ob_acc[r, sl]
                        p = ob_prev[r, sl]
                        y = jnp.maximum(x, LEAK * x) + p
                        ob_prev[r, sl] = y
                        if first_layer:
                            ob_vec[r, sl] = p + y
                        else:
                            ob_vec[r, sl] = ob_vec[r, sl] + y

                if not last_layer:
                    pltpu.sync_copy(ob_prev, tab_out.at[pl.ds(g0, BR)])
                pltpu.sync_copy(ob_vec, vec_out.at[pl.ds(g0, BR)])

            plsc.subcore_barrier()

    do_direction(i_tab, a_src, a_tgt, a_off, u_tab, vec_u_in, u_new, vu_out)
    do_direction(u_tab, t_src, t_tgt, t_off, i_tab, vec_i_in, i_new, vi_out)


_tab = jax.ShapeDtypeStruct((N, D), jnp.float32)

_scratch = [
    pltpu.VMEM_SHARED((R + 8, D), jnp.float32),
    pltpu.VMEM((16,), jnp.int32),
    pltpu.VMEM((SUP * EB,), jnp.int32),
    pltpu.VMEM((SUP * EB,), jnp.int32),
    pltpu.VMEM((EB,), jnp.int32),
    pltpu.VMEM((EB,), jnp.int32),
    pltpu.VMEM((EB,), jnp.int32),
    pltpu.VMEM((EB,), jnp.int32),
    pltpu.VMEM((EB, D), jnp.float32),
    pltpu.VMEM((EB, D), jnp.float32),
    pltpu.VMEM((BR, D), jnp.float32),
    pltpu.VMEM((BR, D), jnp.float32),
    pltpu.VMEM((BR, D), jnp.float32),
    pltpu.VMEM((BR, D), jnp.float32),
    pltpu.SemaphoreType.DMA,
    pltpu.SemaphoreType.DMA,
    pltpu.SemaphoreType.DMA,
]

_mesh = plsc.VectorSubcoreMesh(core_axis_name="c", subcore_axis_name="s")

_layer1 = pl.kernel(
    functools.partial(_layer_body, True, False),
    out_type=(_tab, _tab, _tab, _tab),
    mesh=_mesh,
    scratch_types=_scratch,
)

_layer2 = pl.kernel(
    functools.partial(_layer_body, False, True),
    out_type=(_tab, _tab, _tab, _tab),
    mesh=_mesh,
    scratch_types=_scratch,
)


def _mk_off(tgt):
    bounds = (jnp.arange(1, NCHUNK, dtype=jnp.int32) * R)
    b = jnp.searchsorted(tgt, bounds, side="left").astype(jnp.int32)
    return jnp.concatenate([
        jnp.zeros((1,), jnp.int32), b, jnp.full((1,), E, jnp.int32),
        jnp.zeros((16 - NCHUNK - 1,), jnp.int32)])


def kernel(user_embeddings, item_embeddings, adj_src, adj_tgt,
           tpadj_src, tpadj_tgt):
    user_vecs = []
    item_vecs = []
    for k in range(GRAPHNUM):
        u0 = user_embeddings[k]
        i0 = item_embeddings[k]
        asrc = jnp.pad(adj_src[k], (0, EPAD))
        atgt = jnp.pad(adj_tgt[k], (0, EPAD))
        tsrc = jnp.pad(tpadj_src[k], (0, EPAD))
        ttgt = jnp.pad(tpadj_tgt[k], (0, EPAD))
        a_off = _mk_off(adj_tgt[k])
        t_off = _mk_off(tpadj_tgt[k])
        u1, i1, vu1, vi1 = _layer1(i0, u0, asrc, atgt, tsrc, ttgt,
                                   a_off, t_off, u0, i0)
        _, _, vu2, vi2 = _layer2(i1, u1, asrc, atgt, tsrc, ttgt,
                                 a_off, t_off, vu1, vi1)
        user_vecs.append(vu2)
        item_vecs.append(vi2)
    return (jnp.stack(user_vecs, axis=0), jnp.stack(item_vecs, axis=0))


# speedup vs baseline: 10.6671x; 1.0733x over previous
import functools

import jax
import jax.numpy as jnp
from jax import lax
from jax.experimental import pallas as pl
from jax.experimental.pallas import tpu as pltpu
from jax.experimental.pallas import tpu_sc as plsc

N = 50000
D = 128
E = 600000
GRAPHNUM = 2
NCHUNK = 10
R = N // NCHUNK
BR = 40
NBLK = R // BR
EB = 128
SUP = 16
EPAD = SUP * EB
NSUB = 16
NLANE = 16
LEAK = 0.01


def _layer_body(first_layer, last_layer,
                i_tab, u_tab, a_src, a_tgt, t_src, t_tgt, a_off, t_off,
                vec_u_in, vec_i_in,
                u_new, i_new, vu_out, vi_out,
                acc, offs, sbulk, tbulk, ssm0, ssm1, tsm0, tsm1,
                rows0, rows1, zb,
                oba0, oba1, obp0, obp1, obv0, obv1,
                semb, sem0, sem1, isema0, isema1, isemh0, isemh1,
                osem0, osem1):
    c = lax.axis_index("c")
    s = lax.axis_index("s")
    ssm = (ssm0, ssm1)
    tsm = (tsm0, tsm1)
    rows = (rows0, rows1)
    sems = (sem0, sem1)
    oba = (oba0, oba1)
    obp = (obp0, obp1)
    obv = (obv0, obv1)
    isema = (isema0, isema1)
    isemh = (isemh0, isemh1)
    osem = (osem0, osem1)

    @pl.loop(0, BR)
    def _(r):
        for j in range(D // NLANE):
            zb[r, pl.ds(NLANE * j, NLANE)] = jnp.zeros((NLANE,), jnp.float32)

    def do_direction(src_tab, e_src, e_tgt, off_hbm, prev_tab, vec_in,
                     tab_out, vec_out):
        pltpu.sync_copy(off_hbm, offs)
        ovec = offs[pl.ds(0, 16)]

        for slot in range(NCHUNK // 2):
            chunk = 2 * slot + c
            base = chunk * R
            lo = jnp.where(c == 0, ovec[2 * slot], ovec[2 * slot + 1])
            hi = jnp.where(c == 0, ovec[2 * slot + 1], ovec[2 * slot + 2])

            @pl.loop(s, NBLK, step=NSUB)
            def _(b):
                pltpu.sync_copy(zb, acc.at[pl.ds(b * BR, BR)])

            plsc.subcore_barrier()

            n = hi - lo
            t_lo = lo + (n * s) // NSUB
            t_hi = lo + (n * (s + 1)) // NSUB
            a_lo = (t_lo // 16) * 16
            nb = jnp.maximum((t_hi - a_lo + EB - 1) // EB, 0)

            def bulk_load(bi):
                e = a_lo + bi * EB
                d1 = pltpu.async_copy(e_src.at[pl.ds(e, SUP * EB)], sbulk,
                                      semb)
                d2 = pltpu.async_copy(e_tgt.at[pl.ds(e, SUP * EB)], tbulk,
                                      semb)
                d1.wait()
                d2.wait()

            def stage(bi, par):
                q = lax.rem(bi, SUP)
                e = a_lo + bi * EB
                for j in range(EB // NLANE):
                    sl = pl.ds(NLANE * j, NLANE)
                    bsl = pl.ds(q * EB + NLANE * j, NLANE)
                    pos = e + NLANE * j + lax.iota(jnp.int32, 16)
                    valid = (pos >= t_lo) & (pos < t_hi)
                    ssm[par][sl] = sbulk[bsl]
                    tsm[par][sl] = jnp.where(valid, tbulk[bsl] - base, R)
                pltpu.async_copy(src_tab.at[ssm[par]], rows[par], sems[par])

            @pl.when(nb > 0)
            def _():
                bulk_load(0)
                stage(0, 0)

            @pl.loop(0, (nb + 1) // 2)
            def _(h):
                for par in range(2):
                    bi = 2 * h + par

                    @pl.when(bi < nb)
                    def _():
                        nxt = bi + 1

                        @pl.when(nxt < nb)
                        def _():
                            @pl.when(lax.rem(nxt, SUP) == 0)
                            def _():
                                bulk_load(nxt)

                            stage(nxt, 1 - par)

                        pltpu.make_async_copy(src_tab.at[ssm[par]],
                                              rows[par], sems[par]).wait()
                        pltpu.sync_copy(rows[par], acc.at[tsm[par]], add=True)

            plsc.subcore_barrier()

            def ep_in(b, pp):
                g0 = base + b * BR
                da = pltpu.async_copy(acc.at[pl.ds(b * BR, BR)], oba[pp],
                                      isema[pp])
                dp = pltpu.async_copy(prev_tab.at[pl.ds(g0, BR)], obp[pp],
                                      isemh[pp])
                dv = None
                if not first_layer:
                    dv = pltpu.async_copy(vec_in.at[pl.ds(g0, BR)], obv[pp],
                                          isemh[pp])
                return da, dp, dv

            def ep_wait(ds3):
                da, dp, dv = ds3
                da.wait()
                dp.wait()
                if dv is not None:
                    dv.wait()

            def ep_compute(pp):
                @pl.loop(0, BR)
                def _(r):
                    for j in range(D // NLANE):
                        sl = pl.ds(NLANE * j, NLANE)
                        x = oba[pp][r, sl]
                        pv = obp[pp][r, sl]
                        y = jnp.maximum(x, LEAK * x) + pv
                        obp[pp][r, sl] = y
                        if first_layer:
                            obv[pp][r, sl] = pv + y
                        else:
                            obv[pp][r, sl] = obv[pp][r, sl] + y

            def ep_out(b, pp):
                g0 = base + b * BR
                dp = None
                if not last_layer:
                    dp = pltpu.async_copy(obp[pp], tab_out.at[pl.ds(g0, BR)],
                                          osem[pp])
                dv = pltpu.async_copy(obv[pp], vec_out.at[pl.ds(g0, BR)],
                                      osem[pp])
                return dp, dv

            def ep_wait_out(ds2):
                dp, dv = ds2
                if dp is not None:
                    dp.wait()
                dv.wait()

            @pl.loop(0, (NBLK + 2 * NSUB - 1 - s) // (2 * NSUB))
            def _(hh):
                b0 = s + 2 * hh * NSUB
                b1 = b0 + NSUB

                @pl.when(b1 < NBLK)
                def _():
                    d0 = ep_in(b0, 0)
                    d1 = ep_in(b1, 1)
                    ep_wait(d0)
                    ep_compute(0)
                    o0 = ep_out(b0, 0)
                    ep_wait(d1)
                    ep_compute(1)
                    o1 = ep_out(b1, 1)
                    ep_wait_out(o0)
                    ep_wait_out(o1)

                @pl.when(b1 >= NBLK)
                def _():
                    d0 = ep_in(b0, 0)
                    ep_wait(d0)
                    ep_compute(0)
                    o0 = ep_out(b0, 0)
                    ep_wait_out(o0)

            plsc.subcore_barrier()

    do_direction(i_tab, a_src, a_tgt, a_off, u_tab, vec_u_in, u_new, vu_out)
    do_direction(u_tab, t_src, t_tgt, t_off, i_tab, vec_i_in, i_new, vi_out)


_tab = jax.ShapeDtypeStruct((N, D), jnp.float32)

_scratch = [
    pltpu.VMEM_SHARED((R + 8, D), jnp.float32),
    pltpu.VMEM((16,), jnp.int32),
    pltpu.VMEM((SUP * EB,), jnp.int32),
    pltpu.VMEM((SUP * EB,), jnp.int32),
    pltpu.VMEM((EB,), jnp.int32),
    pltpu.VMEM((EB,), jnp.int32),
    pltpu.VMEM((EB,), jnp.int32),
    pltpu.VMEM((EB,), jnp.int32),
    pltpu.VMEM((EB, D), jnp.float32),
    pltpu.VMEM((EB, D), jnp.float32),
    pltpu.VMEM((BR, D), jnp.float32),
    pltpu.VMEM((BR, D), jnp.float32),
    pltpu.VMEM((BR, D), jnp.float32),
    pltpu.VMEM((BR, D), jnp.float32),
    pltpu.VMEM((BR, D), jnp.float32),
    pltpu.VMEM((BR, D), jnp.float32),
    pltpu.VMEM((BR, D), jnp.float32),
    pltpu.SemaphoreType.DMA,
    pltpu.SemaphoreType.DMA,
    pltpu.SemaphoreType.DMA,
    pltpu.SemaphoreType.DMA,
    pltpu.SemaphoreType.DMA,
    pltpu.SemaphoreType.DMA,
    pltpu.SemaphoreType.DMA,
    pltpu.SemaphoreType.DMA,
    pltpu.SemaphoreType.DMA,
]

_mesh = plsc.VectorSubcoreMesh(core_axis_name="c", subcore_axis_name="s")

_layer1 = pl.kernel(
    functools.partial(_layer_body, True, False),
    out_type=(_tab, _tab, _tab, _tab),
    mesh=_mesh,
    scratch_types=_scratch,
)

_layer2 = pl.kernel(
    functools.partial(_layer_body, False, True),
    out_type=(_tab, _tab, _tab, _tab),
    mesh=_mesh,
    scratch_types=_scratch,
)


def _mk_off(tgt):
    bounds = (jnp.arange(1, NCHUNK, dtype=jnp.int32) * R)
    b = jnp.searchsorted(tgt, bounds, side="left").astype(jnp.int32)
    return jnp.concatenate([
        jnp.zeros((1,), jnp.int32), b, jnp.full((1,), E, jnp.int32),
        jnp.zeros((16 - NCHUNK - 1,), jnp.int32)])


def kernel(user_embeddings, item_embeddings, adj_src, adj_tgt,
           tpadj_src, tpadj_tgt):
    user_vecs = []
    item_vecs = []
    for k in range(GRAPHNUM):
        u0 = user_embeddings[k]
        i0 = item_embeddings[k]
        asrc = jnp.pad(adj_src[k], (0, EPAD))
        atgt = jnp.pad(adj_tgt[k], (0, EPAD))
        tsrc = jnp.pad(tpadj_src[k], (0, EPAD))
        ttgt = jnp.pad(tpadj_tgt[k], (0, EPAD))
        a_off = _mk_off(adj_tgt[k])
        t_off = _mk_off(tpadj_tgt[k])
        u1, i1, vu1, vi1 = _layer1(i0, u0, asrc, atgt, tsrc, ttgt,
                                   a_off, t_off, u0, i0)
        _, _, vu2, vi2 = _layer2(i1, u1, asrc, atgt, tsrc, ttgt,
                                 a_off, t_off, vu1, vi1)
        user_vecs.append(vu2)
        item_vecs.append(vi2)
    return (jnp.stack(user_vecs, axis=0), jnp.stack(item_vecs, axis=0))


# speedup vs baseline: 11.1428x; 1.0446x over previous
import functools

import jax
import jax.numpy as jnp
from jax import lax
from jax.experimental import pallas as pl
from jax.experimental.pallas import tpu as pltpu
from jax.experimental.pallas import tpu_sc as plsc

N = 50000
D = 128
E = 600000
GRAPHNUM = 2
NCHUNK = 10
R = N // NCHUNK
BR = 40
NBLK = R // BR
EB = 128
SUP = 16
EPAD = SUP * EB
NSUB = 16
NLANE = 16
LEAK = 0.01


def _layer_body(first_layer, last_layer,
                i_tab, u_tab, a_src, a_tgt, t_src, t_tgt, a_off, t_off,
                vec_u_in, vec_i_in,
                u_new, i_new, vu_out, vi_out,
                acc, offs, sbulk, tbulk, ssm0, ssm1, ssm2,
                tsm0, tsm1, tsm2,
                rows0, rows1, rows2, zb,
                oba0, oba1, obp0, obp1, obv0, obv1,
                semb, sem0, sem1, sem2, isema0, isema1, isemh0, isemh1,
                osem0, osem1):
    c = lax.axis_index("c")
    s = lax.axis_index("s")
    ssm = (ssm0, ssm1, ssm2)
    tsm = (tsm0, tsm1, tsm2)
    rows = (rows0, rows1, rows2)
    sems = (sem0, sem1, sem2)
    oba = (oba0, oba1)
    obp = (obp0, obp1)
    obv = (obv0, obv1)
    isema = (isema0, isema1)
    isemh = (isemh0, isemh1)
    osem = (osem0, osem1)

    @pl.loop(0, BR)
    def _(r):
        for j in range(D // NLANE):
            zb[r, pl.ds(NLANE * j, NLANE)] = jnp.zeros((NLANE,), jnp.float32)

    def do_direction(src_tab, e_src, e_tgt, off_hbm, prev_tab, vec_in,
                     tab_out, vec_out):
        pltpu.sync_copy(off_hbm, offs)
        ovec = offs[pl.ds(0, 16)]

        for slot in range(NCHUNK // 2):
            chunk = 2 * slot + c
            base = chunk * R
            lo = jnp.where(c == 0, ovec[2 * slot], ovec[2 * slot + 1])
            hi = jnp.where(c == 0, ovec[2 * slot + 1], ovec[2 * slot + 2])

            @pl.loop(s, NBLK, step=NSUB)
            def _(b):
                pltpu.sync_copy(zb, acc.at[pl.ds(b * BR, BR)])

            plsc.subcore_barrier()

            n = hi - lo
            t_lo = lo + (n * s) // NSUB
            t_hi = lo + (n * (s + 1)) // NSUB
            a_lo = (t_lo // 16) * 16
            nb = jnp.maximum((t_hi - a_lo + EB - 1) // EB, 0)

            def bulk_load(bi):
                e = a_lo + bi * EB
                d1 = pltpu.async_copy(e_src.at[pl.ds(e, SUP * EB)], sbulk,
                                      semb)
                d2 = pltpu.async_copy(e_tgt.at[pl.ds(e, SUP * EB)], tbulk,
                                      semb)
                d1.wait()
                d2.wait()

            def stage(bi, par):
                q = lax.rem(bi, SUP)
                e = a_lo + bi * EB
                for j in range(EB // NLANE):
                    sl = pl.ds(NLANE * j, NLANE)
                    bsl = pl.ds(q * EB + NLANE * j, NLANE)
                    pos = e + NLANE * j + lax.iota(jnp.int32, 16)
                    valid = (pos >= t_lo) & (pos < t_hi)
                    ssm[par][sl] = sbulk[bsl]
                    tsm[par][sl] = jnp.where(valid, tbulk[bsl] - base, R)
                pltpu.async_copy(src_tab.at[ssm[par]], rows[par], sems[par])

            @pl.when(nb > 0)
            def _():
                bulk_load(0)
                stage(0, 0)

            @pl.when(nb > 1)
            def _():
                stage(1, 1)

            @pl.loop(0, (nb + 2) // 3)
            def _(h):
                for par in range(3):
                    bi = 3 * h + par

                    @pl.when(bi < nb)
                    def _():
                        nxt = bi + 2

                        @pl.when(nxt < nb)
                        def _():
                            @pl.when(lax.rem(nxt, SUP) == 0)
                            def _():
                                bulk_load(nxt)

                            stage(nxt, (par + 2) % 3)

                        pltpu.make_async_copy(src_tab.at[ssm[par]],
                                              rows[par], sems[par]).wait()
                        pltpu.sync_copy(rows[par], acc.at[tsm[par]], add=True)

            plsc.subcore_barrier()

            def ep_in(b, pp):
                g0 = base + b * BR
                da = pltpu.async_copy(acc.at[pl.ds(b * BR, BR)], oba[pp],
                                      isema[pp])
                dp = pltpu.async_copy(prev_tab.at[pl.ds(g0, BR)], obp[pp],
                                      isemh[pp])
                dv = None
                if not first_layer:
                    dv = pltpu.async_copy(vec_in.at[pl.ds(g0, BR)], obv[pp],
                                          isemh[pp])
                return da, dp, dv

            def ep_wait(ds3):
                da, dp, dv = ds3
                da.wait()
                dp.wait()
                if dv is not None:
                    dv.wait()

            def ep_compute(pp):
                @pl.loop(0, BR)
                def _(r):
                    for j in range(D // NLANE):
                        sl = pl.ds(NLANE * j, NLANE)
                        x = oba[pp][r, sl]
                        pv = obp[pp][r, sl]
                        y = jnp.maximum(x, LEAK * x) + pv
                        obp[pp][r, sl] = y
                        if first_layer:
                            obv[pp][r, sl] = pv + y
                        else:
                            obv[pp][r, sl] = obv[pp][r, sl] + y

            def ep_out(b, pp):
                g0 = base + b * BR
                dp = None
                if not last_layer:
                    dp = pltpu.async_copy(obp[pp], tab_out.at[pl.ds(g0, BR)],
                                          osem[pp])
                dv = pltpu.async_copy(obv[pp], vec_out.at[pl.ds(g0, BR)],
                                      osem[pp])
                return dp, dv

            def ep_wait_out(ds2):
                dp, dv = ds2
                if dp is not None:
                    dp.wait()
                dv.wait()

            @pl.loop(0, (NBLK + 2 * NSUB - 1 - s) // (2 * NSUB))
            def _(hh):
                b0 = s + 2 * hh * NSUB
                b1 = b0 + NSUB

                @pl.when(b1 < NBLK)
                def _():
                    d0 = ep_in(b0, 0)
                    d1 = ep_in(b1, 1)
                    ep_wait(d0)
                    ep_compute(0)
                    o0 = ep_out(b0, 0)
                    ep_wait(d1)
                    ep_compute(1)
                    o1 = ep_out(b1, 1)
                    ep_wait_out(o0)
                    ep_wait_out(o1)

                @pl.when(b1 >= NBLK)
                def _():
                    d0 = ep_in(b0, 0)
                    ep_wait(d0)
                    ep_compute(0)
                    o0 = ep_out(b0, 0)
                    ep_wait_out(o0)

            plsc.subcore_barrier()

    do_direction(i_tab, a_src, a_tgt, a_off, u_tab, vec_u_in, u_new, vu_out)
    do_direction(u_tab, t_src, t_tgt, t_off, i_tab, vec_i_in, i_new, vi_out)


_tab = jax.ShapeDtypeStruct((N, D), jnp.float32)

_scratch = [
    pltpu.VMEM_SHARED((R + 8, D), jnp.float32),
    pltpu.VMEM((16,), jnp.int32),
    pltpu.VMEM((SUP * EB,), jnp.int32),
    pltpu.VMEM((SUP * EB,), jnp.int32),
    pltpu.VMEM((EB,), jnp.int32),
    pltpu.VMEM((EB,), jnp.int32),
    pltpu.VMEM((EB,), jnp.int32),
    pltpu.VMEM((EB,), jnp.int32),
    pltpu.VMEM((EB,), jnp.int32),
    pltpu.VMEM((EB,), jnp.int32),
    pltpu.VMEM((EB, D), jnp.float32),
    pltpu.VMEM((EB, D), jnp.float32),
    pltpu.VMEM((EB, D), jnp.float32),
    pltpu.VMEM((BR, D), jnp.float32),
    pltpu.VMEM((BR, D), jnp.float32),
    pltpu.VMEM((BR, D), jnp.float32),
    pltpu.VMEM((BR, D), jnp.float32),
    pltpu.VMEM((BR, D), jnp.float32),
    pltpu.VMEM((BR, D), jnp.float32),
    pltpu.VMEM((BR, D), jnp.float32),
    pltpu.SemaphoreType.DMA,
    pltpu.SemaphoreType.DMA,
    pltpu.SemaphoreType.DMA,
    pltpu.SemaphoreType.DMA,
    pltpu.SemaphoreType.DMA,
    pltpu.SemaphoreType.DMA,
    pltpu.SemaphoreType.DMA,
    pltpu.SemaphoreType.DMA,
    pltpu.SemaphoreType.DMA,
    pltpu.SemaphoreType.DMA,
]

_mesh = plsc.VectorSubcoreMesh(core_axis_name="c", subcore_axis_name="s")

_layer1 = pl.kernel(
    functools.partial(_layer_body, True, False),
    out_type=(_tab, _tab, _tab, _tab),
    mesh=_mesh,
    scratch_types=_scratch,
)

_layer2 = pl.kernel(
    functools.partial(_layer_body, False, True),
    out_type=(_tab, _tab, _tab, _tab),
    mesh=_mesh,
    scratch_types=_scratch,
)


def _mk_off(tgt):
    bounds = (jnp.arange(1, NCHUNK, dtype=jnp.int32) * R)
    b = jnp.searchsorted(tgt, bounds, side="left").astype(jnp.int32)
    return jnp.concatenate([
        jnp.zeros((1,), jnp.int32), b, jnp.full((1,), E, jnp.int32),
        jnp.zeros((16 - NCHUNK - 1,), jnp.int32)])


def kernel(user_embeddings, item_embeddings, adj_src, adj_tgt,
           tpadj_src, tpadj_tgt):
    user_vecs = []
    item_vecs = []
    for k in range(GRAPHNUM):
        u0 = user_embeddings[k]
        i0 = item_embeddings[k]
        asrc = jnp.pad(adj_src[k], (0, EPAD))
        atgt = jnp.pad(adj_tgt[k], (0, EPAD))
        tsrc = jnp.pad(tpadj_src[k], (0, EPAD))
        ttgt = jnp.pad(tpadj_tgt[k], (0, EPAD))
        a_off = _mk_off(adj_tgt[k])
        t_off = _mk_off(tpadj_tgt[k])
        u1, i1, vu1, vi1 = _layer1(i0, u0, asrc, atgt, tsrc, ttgt,
                                   a_off, t_off, u0, i0)
        _, _, vu2, vi2 = _layer2(i1, u1, asrc, atgt, tsrc, ttgt,
                                 a_off, t_off, vu1, vi1)
        user_vecs.append(vu2)
        item_vecs.append(vi2)
    return (jnp.stack(user_vecs, axis=0), jnp.stack(item_vecs, axis=0))


# speedup vs baseline: 11.1647x; 1.0020x over previous
import functools

import jax
import jax.numpy as jnp
from jax import lax
from jax.experimental import pallas as pl
from jax.experimental.pallas import tpu as pltpu
from jax.experimental.pallas import tpu_sc as plsc

N = 50000
D = 128
E = 600000
GRAPHNUM = 2
NCHUNK = 10
R = N // NCHUNK
BR = 40
NBLK = R // BR
EB = 128
SUP = 16
EPAD = SUP * EB
NSUB = 16
NLANE = 16
LEAK = 0.01


def _layer_body(first_layer, last_layer,
                i_tab, u_tab, a_src, a_tgt, t_src, t_tgt, a_off, t_off,
                vec_u_in, vec_i_in,
                u_new, i_new, vu_out, vi_out,
                acc, offs, sbulk, tbulk, ssm0, ssm1, ssm2,
                tsm0, tsm1, tsm2,
                rows0, rows1, rows2, zb,
                oba0, oba1, obp0, obp1, obv0, obv1,
                semb, sem0, sem1, sem2, ssem0, ssem1, ssem2,
                isema0, isema1, isemh0, isemh1,
                osem0, osem1):
    c = lax.axis_index("c")
    s = lax.axis_index("s")
    ssm = (ssm0, ssm1, ssm2)
    tsm = (tsm0, tsm1, tsm2)
    rows = (rows0, rows1, rows2)
    sems = (sem0, sem1, sem2)
    ssem = (ssem0, ssem1, ssem2)
    oba = (oba0, oba1)
    obp = (obp0, obp1)
    obv = (obv0, obv1)
    isema = (isema0, isema1)
    isemh = (isemh0, isemh1)
    osem = (osem0, osem1)

    @pl.loop(0, BR)
    def _(r):
        for j in range(D // NLANE):
            zb[r, pl.ds(NLANE * j, NLANE)] = jnp.zeros((NLANE,), jnp.float32)

    def do_direction(src_tab, e_src, e_tgt, off_hbm, prev_tab, vec_in,
                     tab_out, vec_out):
        pltpu.sync_copy(off_hbm, offs)
        ovec = offs[pl.ds(0, 16)]

        for slot in range(NCHUNK // 2):
            chunk = 2 * slot + c
            base = chunk * R
            lo = jnp.where(c == 0, ovec[2 * slot], ovec[2 * slot + 1])
            hi = jnp.where(c == 0, ovec[2 * slot + 1], ovec[2 * slot + 2])

            @pl.loop(s, NBLK, step=NSUB)
            def _(b):
                pltpu.sync_copy(zb, acc.at[pl.ds(b * BR, BR)])

            plsc.subcore_barrier()

            n = hi - lo
            t_lo = lo + (n * s) // NSUB
            t_hi = lo + (n * (s + 1)) // NSUB
            a_lo = (t_lo // 16) * 16
            nb = jnp.maximum((t_hi - a_lo + EB - 1) // EB, 0)

            def bulk_load(bi):
                e = a_lo + bi * EB
                d1 = pltpu.async_copy(e_src.at[pl.ds(e, SUP * EB)], sbulk,
                                      semb)
                d2 = pltpu.async_copy(e_tgt.at[pl.ds(e, SUP * EB)], tbulk,
                                      semb)
                d1.wait()
                d2.wait()

            def stage(bi, par):
                q = lax.rem(bi, SUP)
                e = a_lo + bi * EB
                for j in range(EB // NLANE):
                    sl = pl.ds(NLANE * j, NLANE)
                    bsl = pl.ds(q * EB + NLANE * j, NLANE)
                    pos = e + NLANE * j + lax.iota(jnp.int32, 16)
                    valid = (pos >= t_lo) & (pos < t_hi)
                    ssm[par][sl] = sbulk[bsl]
                    tsm[par][sl] = jnp.where(valid, tbulk[bsl] - base, R)
                pltpu.async_copy(src_tab.at[ssm[par]], rows[par], sems[par])

            @pl.when(nb > 0)
            def _():
                bulk_load(0)
                stage(0, 0)

            @pl.when(nb > 1)
            def _():
                stage(1, 1)

            def wait_scatter(p):
                pltpu.make_async_copy(rows[p], acc.at[tsm[p]],
                                      ssem[p]).wait()

            @pl.loop(0, (nb + 2) // 3)
            def _(h):
                for par in range(3):
                    bi = 3 * h + par

                    @pl.when(bi < nb)
                    def _():
                        nxt = bi + 2

                        @pl.when(nxt < nb)
                        def _():
                            @pl.when(bi >= 1)
                            def _():
                                wait_scatter((par + 2) % 3)

                            @pl.when(lax.rem(nxt, SUP) == 0)
                            def _():
                                bulk_load(nxt)

                            stage(nxt, (par + 2) % 3)

                        pltpu.make_async_copy(src_tab.at[ssm[par]],
                                              rows[par], sems[par]).wait()
                        pltpu.async_copy(rows[par], acc.at[tsm[par]],
                                         ssem[par], add=True)

            @pl.when(nb >= 3)
            def _():
                wait_scatter(0)
                wait_scatter(1)
                wait_scatter(2)

            @pl.when(nb == 2)
            def _():
                wait_scatter(0)
                wait_scatter(1)

            @pl.when(nb == 1)
            def _():
                wait_scatter(0)

            plsc.subcore_barrier()

            def ep_in(b, pp):
                g0 = base + b * BR
                da = pltpu.async_copy(acc.at[pl.ds(b * BR, BR)], oba[pp],
                                      isema[pp])
                dp = pltpu.async_copy(prev_tab.at[pl.ds(g0, BR)], obp[pp],
                                      isemh[pp])
                dv = None
                if not first_layer:
                    dv = pltpu.async_copy(vec_in.at[pl.ds(g0, BR)], obv[pp],
                                          isemh[pp])
                return da, dp, dv

            def ep_wait(ds3):
                da, dp, dv = ds3
                da.wait()
                dp.wait()
                if dv is not None:
                    dv.wait()

            def ep_compute(pp):
                @pl.loop(0, BR)
                def _(r):
                    for j in range(D // NLANE):
                        sl = pl.ds(NLANE * j, NLANE)
                        x = oba[pp][r, sl]
                        pv = obp[pp][r, sl]
                        y = jnp.maximum(x, LEAK * x) + pv
                        obp[pp][r, sl] = y
                        if first_layer:
                            obv[pp][r, sl] = pv + y
                        else:
                            obv[pp][r, sl] = obv[pp][r, sl] + y

            def ep_out(b, pp):
                g0 = base + b * BR
                dp = None
                if not last_layer:
                    dp = pltpu.async_copy(obp[pp], tab_out.at[pl.ds(g0, BR)],
                                          osem[pp])
                dv = pltpu.async_copy(obv[pp], vec_out.at[pl.ds(g0, BR)],
                                      osem[pp])
                return dp, dv

            def ep_wait_out(ds2):
                dp, dv = ds2
                if dp is not None:
                    dp.wait()
                dv.wait()

            @pl.loop(0, (NBLK + 2 * NSUB - 1 - s) // (2 * NSUB))
            def _(hh):
                b0 = s + 2 * hh * NSUB
                b1 = b0 + NSUB

                @pl.when(b1 < NBLK)
                def _():
                    d0 = ep_in(b0, 0)
                    d1 = ep_in(b1, 1)
                    ep_wait(d0)
                    ep_compute(0)
                    o0 = ep_out(b0, 0)
                    ep_wait(d1)
                    ep_compute(1)
                    o1 = ep_out(b1, 1)
                    ep_wait_out(o0)
                    ep_wait_out(o1)

                @pl.when(b1 >= NBLK)
                def _():
                    d0 = ep_in(b0, 0)
                    ep_wait(d0)
                    ep_compute(0)
                    o0 = ep_out(b0, 0)
                    ep_wait_out(o0)

            plsc.subcore_barrier()

    do_direction(i_tab, a_src, a_tgt, a_off, u_tab, vec_u_in, u_new, vu_out)
    do_direction(u_tab, t_src, t_tgt, t_off, i_tab, vec_i_in, i_new, vi_out)


_tab = jax.ShapeDtypeStruct((N, D), jnp.float32)

_scratch = [
    pltpu.VMEM_SHARED((R + 8, D), jnp.float32),
    pltpu.VMEM((16,), jnp.int32),
    pltpu.VMEM((SUP * EB,), jnp.int32),
    pltpu.VMEM((SUP * EB,), jnp.int32),
    pltpu.VMEM((EB,), jnp.int32),
    pltpu.VMEM((EB,), jnp.int32),
    pltpu.VMEM((EB,), jnp.int32),
    pltpu.VMEM((EB,), jnp.int32),
    pltpu.VMEM((EB,), jnp.int32),
    pltpu.VMEM((EB,), jnp.int32),
    pltpu.VMEM((EB, D), jnp.float32),
    pltpu.VMEM((EB, D), jnp.float32),
    pltpu.VMEM((EB, D), jnp.float32),
    pltpu.VMEM((BR, D), jnp.float32),
    pltpu.VMEM((BR, D), jnp.float32),
    pltpu.VMEM((BR, D), jnp.float32),
    pltpu.VMEM((BR, D), jnp.float32),
    pltpu.VMEM((BR, D), jnp.float32),
    pltpu.VMEM((BR, D), jnp.float32),
    pltpu.VMEM((BR, D), jnp.float32),
    pltpu.SemaphoreType.DMA,
    pltpu.SemaphoreType.DMA,
    pltpu.SemaphoreType.DMA,
    pltpu.SemaphoreType.DMA,
    pltpu.SemaphoreType.DMA,
    pltpu.SemaphoreType.DMA,
    pltpu.SemaphoreType.DMA,
    pltpu.SemaphoreType.DMA,
    pltpu.SemaphoreType.DMA,
    pltpu.SemaphoreType.DMA,
    pltpu.SemaphoreType.DMA,
    pltpu.SemaphoreType.DMA,
    pltpu.SemaphoreType.DMA,
]

_mesh = plsc.VectorSubcoreMesh(core_axis_name="c", subcore_axis_name="s")

_layer1 = pl.kernel(
    functools.partial(_layer_body, True, False),
    out_type=(_tab, _tab, _tab, _tab),
    mesh=_mesh,
    scratch_types=_scratch,
)

_layer2 = pl.kernel(
    functools.partial(_layer_body, False, True),
    out_type=(_tab, _tab, _tab, _tab),
    mesh=_mesh,
    scratch_types=_scratch,
)


def _mk_off(tgt):
    bounds = (jnp.arange(1, NCHUNK, dtype=jnp.int32) * R)
    b = jnp.searchsorted(tgt, bounds, side="left").astype(jnp.int32)
    return jnp.concatenate([
        jnp.zeros((1,), jnp.int32), b, jnp.full((1,), E, jnp.int32),
        jnp.zeros((16 - NCHUNK - 1,), jnp.int32)])


def kernel(user_embeddings, item_embeddings, adj_src, adj_tgt,
           tpadj_src, tpadj_tgt):
    user_vecs = []
    item_vecs = []
    for k in range(GRAPHNUM):
        u0 = user_embeddings[k]
        i0 = item_embeddings[k]
        asrc = jnp.pad(adj_src[k], (0, EPAD))
        atgt = jnp.pad(adj_tgt[k], (0, EPAD))
        tsrc = jnp.pad(tpadj_src[k], (0, EPAD))
        ttgt = jnp.pad(tpadj_tgt[k], (0, EPAD))
        a_off = _mk_off(adj_tgt[k])
        t_off = _mk_off(tpadj_tgt[k])
        u1, i1, vu1, vi1 = _layer1(i0, u0, asrc, atgt, tsrc, ttgt,
                                   a_off, t_off, u0, i0)
        _, _, vu2, vi2 = _layer2(i1, u1, asrc, atgt, tsrc, ttgt,
                                 a_off, t_off, vu1, vi1)
        user_vecs.append(vu2)
        item_vecs.append(vi2)
    return (jnp.stack(user_vecs, axis=0), jnp.stack(item_vecs, axis=0))
